# Initial kernel scaffold; baseline (speedup 1.0000x reference)
import jax
import jax.numpy as jnp
from jax.experimental import pallas as pl


def kernel(n, embeddingss, tokenEmbeddingss, tokenMaskss, B_diag1, B_diag2):
    raise NotImplementedError("write your pallas kernel here")


# speedup vs baseline: 1.6285x; 1.6285x over previous
import jax
import jax.numpy as jnp
from jax.experimental import pallas as pl

_N_CANDS = 8
_D = 300
_WIN = 50
_ATT_K = 25
_BN = 128


def _body(emb_ref, tok_ref, maskf_ref, b2_ref, b1_ref, out_ref):
    emb = emb_ref[...]
    tok = tok_ref[...]
    maskf = maskf_ref[...]
    b2 = b2_ref[...][:, None, :]
    b1 = b1_ref[...][:, None, :]

    dn = (((2,), (2,)), ((0,), (0,)))
    s = jax.lax.dot_general(emb * b2, tok, dn,
                            preferred_element_type=jnp.float32)
    g = jax.lax.dot_general(emb * b1, tok, dn,
                            preferred_element_type=jnp.float32)
    s = jnp.where(maskf[:, None, :] > 0, s, -1e10)
    ts = jnp.max(s, axis=1)

    col = jax.lax.broadcasted_iota(jnp.int32, (1, _WIN), 1)
    cnt = jnp.zeros_like(ts)
    for k in range(_WIN):
        sk = ts[:, k:k + 1]
        gt = (sk > ts).astype(jnp.float32)
        tie = ((sk == ts) & (k < col)).astype(jnp.float32)
        cnt = cnt + gt + tie
    keep = (cnt < _ATT_K).astype(jnp.float32)

    m = jnp.max(ts, axis=1, keepdims=True)
    e = jnp.exp(ts - m) * keep
    z = jnp.sum(e, axis=1, keepdims=True)
    p = e / z

    out_ref[...] = jnp.sum(g * p[:, None, :], axis=2)


def kernel(n, embeddingss, tokenEmbeddingss, tokenMaskss, B_diag1, B_diag2):
    nm, nc, d = embeddingss.shape
    win = tokenEmbeddingss.shape[1]
    maskf = tokenMaskss.astype(jnp.float32)
    b1 = B_diag1.reshape(1, d)
    b2 = B_diag2.reshape(1, d)
    grid = (nm // _BN,)
    out = pl.pallas_call(
        _body,
        grid=grid,
        in_specs=[
            pl.BlockSpec((_BN, nc, d), lambda i: (i, 0, 0)),
            pl.BlockSpec((_BN, win, d), lambda i: (i, 0, 0)),
            pl.BlockSpec((_BN, win), lambda i: (i, 0)),
            pl.BlockSpec((1, d), lambda i: (0, 0)),
            pl.BlockSpec((1, d), lambda i: (0, 0)),
        ],
        out_specs=pl.BlockSpec((_BN, nc), lambda i: (i, 0)),
        out_shape=jax.ShapeDtypeStruct((nm, nc), jnp.float32),
    )(embeddingss, tokenEmbeddingss, maskf, b2, b1)
    return out
